# baseline (speedup 1.0000x reference)
import functools

import jax
import jax.numpy as jnp
from jax import lax
from jax.experimental import pallas as pl
from jax.experimental.pallas import tpu as pltpu

K = 16
B1 = 128
CC = 512
BI = 256
BQ = 128


def _phase1a_body(x_ref, b_ref, xq_ref, bq_ref, dt_ref, t16_ref):
    c = pl.program_id(0)
    cc = pl.program_id(1)
    base = c * B1
    s = cc * CC
    xq = xq_ref[:, :]
    sq_q = jnp.sum(xq * xq, axis=1)
    bq = bq_ref[0, :]
    xc = x_ref[:, :]
    sq_c = jnp.sum(xc * xc, axis=1)
    bc = b_ref[0, :]
    mmt = lax.dot_general(xc, xq, (((1,), (1,)), ((), ())),
                          preferred_element_type=jnp.float32)
    dt = sq_q[None, :] - 2.0 * mmt + sq_c[:, None]
    ti = lax.broadcasted_iota(jnp.int32, (CC, B1), 0) + s
    tq = lax.broadcasted_iota(jnp.int32, (CC, B1), 1) + base
    dt = jnp.where((bc[:, None] != bq[None, :]) | (ti == tq), jnp.inf, dt)
    dt_ref[:, :] = dt

    ri = lax.broadcasted_iota(jnp.int32, (CC, B1), 0)
    big = jnp.int32(2**30)

    def step(t, carry):
        v, acc = carry
        m = jnp.min(v, axis=0)
        pos = jnp.min(jnp.where(v == m[None, :], ri, big), axis=0)
        v = jnp.where(ri == pos[None, :], jnp.inf, v)
        acc = jnp.where(
            lax.broadcasted_iota(jnp.int32, (K, B1), 0) == t, m[None, :], acc)
        return v, acc

    _, t16 = lax.fori_loop(
        0, K, step, (dt, jnp.full((K, B1), jnp.inf, dtype=jnp.float32)))
    t16_ref[0, :, :] = t16


def _phase1b_body(nch, t16_ref, tau_ref):
    v = t16_ref[:, :, :].reshape(nch * K, B1)
    ri = lax.broadcasted_iota(jnp.int32, (nch * K, B1), 0)
    big = jnp.int32(2**30)

    def step(t, vv):
        m = jnp.min(vv, axis=0)
        pos = jnp.min(jnp.where(vv == m[None, :], ri, big), axis=0)
        return jnp.where(ri == pos[None, :], jnp.inf, vv)

    v = lax.fori_loop(0, K - 1, step, v)
    tau_ref[0, :] = jnp.min(v, axis=0)


def _phase2_body(x_ref, tau_ref, qlo_ref, qhi_ref, dt_ref, out_ref):
    c = pl.program_id(0)
    qc = pl.program_id(1)

    @pl.when(qc == 0)
    def _init():
        out_ref[:, :] = jnp.full((BI, 128), -jnp.inf, dtype=jnp.float32)

    qbase = qc * BQ
    active = (qbase < qhi_ref[c]) & (qbase + BQ > qlo_ref[c])

    @pl.when(active)
    def _accum():
        dblk = dt_ref[:, :]
        tau = tau_ref[0, :]
        maskt = dblk <= tau[None, :]
        xq = x_ref[:, :]
        acc = out_ref[:, :]
        for q in range(BQ):
            acc = jnp.maximum(
                acc, jnp.where(maskt[:, q:q + 1], xq[q:q + 1, :], -jnp.inf))
        out_ref[:, :] = acc


def _phase3_body(p_ref, w_ref, b_ref, g_ref, be_ref, out_ref):
    p = p_ref[:, :]
    p = jnp.where(p == -jnp.inf, 0.0, p)
    h = lax.dot_general(p, w_ref[:, :], (((1,), (1,)), ((), ())),
                        preferred_element_type=jnp.float32) + b_ref[0, :][None, :]
    mean = jnp.mean(h, axis=0)
    var = jnp.mean((h - mean[None, :]) ** 2, axis=0)
    h = (h - mean[None, :]) / jnp.sqrt(var + 1e-5) * g_ref[0, :][None, :] \
        + be_ref[0, :][None, :]
    out_ref[:, :] = jnp.maximum(h, 0.0)


@jax.jit
def kernel(x, batch, W, b, gamma, beta):
    n, dim = x.shape
    nq = ((n + CC - 1) // CC) * CC
    g1 = nq // B1
    nch = nq // CC
    gi = nq // BI
    gq = nq // BQ
    xp = jnp.pad(x, ((0, nq - n), (0, 0)))
    bi32 = batch.astype(jnp.int32)
    bp = jnp.pad(bi32, (0, nq - n), constant_values=-1)[None, :]

    dt, t16 = pl.pallas_call(
        _phase1a_body,
        grid=(g1, nch),
        in_specs=[
            pl.BlockSpec((CC, dim), lambda c, cc: (cc, 0)),
            pl.BlockSpec((1, CC), lambda c, cc: (0, cc)),
            pl.BlockSpec((B1, dim), lambda c, cc: (c, 0)),
            pl.BlockSpec((1, B1), lambda c, cc: (0, c)),
        ],
        out_specs=[
            pl.BlockSpec((CC, B1), lambda c, cc: (cc, c)),
            pl.BlockSpec((1, K, B1), lambda c, cc: (cc, 0, c)),
        ],
        out_shape=[
            jax.ShapeDtypeStruct((nq, nq), jnp.float32),
            jax.ShapeDtypeStruct((nch, K, nq), jnp.float32),
        ],
    )(xp, bp, xp, bp)

    tau = pl.pallas_call(
        functools.partial(_phase1b_body, nch),
        grid=(g1,),
        in_specs=[pl.BlockSpec((nch, K, B1), lambda c: (0, 0, c))],
        out_specs=pl.BlockSpec((1, B1), lambda c: (0, c)),
        out_shape=jax.ShapeDtypeStruct((1, nq), jnp.float32),
    )(t16)

    seg_lo = jnp.searchsorted(bi32, jnp.arange(8, dtype=jnp.int32),
                              side='left').astype(jnp.int32)
    seg_hi = jnp.searchsorted(bi32, jnp.arange(8, dtype=jnp.int32),
                              side='right').astype(jnp.int32)
    r0 = jnp.minimum(jnp.arange(gi, dtype=jnp.int32) * BI, n - 1)
    r1 = jnp.minimum(r0 + BI - 1, n - 1)
    qlo = seg_lo[jnp.clip(bi32[r0], 0, 7)]
    qhi = seg_hi[jnp.clip(bi32[r1], 0, 7)]

    pool = pl.pallas_call(
        _phase2_body,
        grid=(gi, gq),
        in_specs=[
            pl.BlockSpec((BQ, dim), lambda c, qc: (qc, 0)),
            pl.BlockSpec((1, BQ), lambda c, qc: (0, qc)),
            pl.BlockSpec(memory_space=pltpu.SMEM),
            pl.BlockSpec(memory_space=pltpu.SMEM),
            pl.BlockSpec((BI, BQ), lambda c, qc: (c, qc)),
        ],
        out_specs=pl.BlockSpec((BI, dim), lambda c, qc: (c, 0)),
        out_shape=jax.ShapeDtypeStruct((nq, dim), jnp.float32),
    )(xp, tau, qlo, qhi, dt)

    out = pl.pallas_call(
        _phase3_body,
        in_specs=[
            pl.BlockSpec((n, dim), lambda: (0, 0)),
            pl.BlockSpec((W.shape[0], dim), lambda: (0, 0)),
            pl.BlockSpec((1, W.shape[0]), lambda: (0, 0)),
            pl.BlockSpec((1, W.shape[0]), lambda: (0, 0)),
            pl.BlockSpec((1, W.shape[0]), lambda: (0, 0)),
        ],
        out_specs=pl.BlockSpec((n, W.shape[0]), lambda: (0, 0)),
        out_shape=jax.ShapeDtypeStruct((n, W.shape[0]), jnp.float32),
    )(pool[:n], W, b[None, :], gamma[None, :], beta[None, :])
    return out


# speedup vs baseline: 3.3642x; 3.3642x over previous
import functools

import jax
import jax.numpy as jnp
from jax import lax
from jax.experimental import pallas as pl
from jax.experimental.pallas import tpu as pltpu

K = 16
B1 = 128
CC = 512
BI = 256
BQ = 128


def _phase1a_body(x_ref, b_ref, xq_ref, bq_ref, qlo_ref, qhi_ref,
                  dt_ref, t16_ref):
    c = pl.program_id(0)
    cc = pl.program_id(1)
    base = c * B1
    s = cc * CC
    active = (s < qhi_ref[c]) & (s + CC > qlo_ref[c])

    @pl.when(jnp.logical_not(active))
    def _skip():
        dt_ref[:, :] = jnp.full((CC, B1), jnp.inf, dtype=jnp.float32)
        t16_ref[0, :, :] = jnp.full((K, B1), jnp.inf, dtype=jnp.float32)

    @pl.when(active)
    def _compute():
        xq = xq_ref[:, :]
        sq_q = jnp.sum(xq * xq, axis=1)
        bq = bq_ref[0, :]
        xc = x_ref[:, :]
        sq_c = jnp.sum(xc * xc, axis=1)
        bc = b_ref[0, :]
        mmt = lax.dot_general(xc, xq, (((1,), (1,)), ((), ())),
                              preferred_element_type=jnp.float32)
        dt = sq_q[None, :] - 2.0 * mmt + sq_c[:, None]
        ti = lax.broadcasted_iota(jnp.int32, (CC, B1), 0) + s
        tq = lax.broadcasted_iota(jnp.int32, (CC, B1), 1) + base
        dt = jnp.where((bc[:, None] != bq[None, :]) | (ti == tq), jnp.inf, dt)
        dt_ref[:, :] = dt

        ri = lax.broadcasted_iota(jnp.int32, (CC, B1), 0)
        big = jnp.int32(2**30)

        def step(t, carry):
            v, acc = carry
            m = jnp.min(v, axis=0)
            pos = jnp.min(jnp.where(v == m[None, :], ri, big), axis=0)
            v = jnp.where(ri == pos[None, :], jnp.inf, v)
            acc = jnp.where(
                lax.broadcasted_iota(jnp.int32, (K, B1), 0) == t,
                m[None, :], acc)
            return v, acc

        _, t16 = lax.fori_loop(
            0, K, step, (dt, jnp.full((K, B1), jnp.inf, dtype=jnp.float32)))
        t16_ref[0, :, :] = t16


def _phase1b_body(nch, t16_ref, tau_ref):
    v = t16_ref[:, :, :].reshape(nch * K, B1)
    ri = lax.broadcasted_iota(jnp.int32, (nch * K, B1), 0)
    big = jnp.int32(2**30)

    def step(t, vv):
        m = jnp.min(vv, axis=0)
        pos = jnp.min(jnp.where(vv == m[None, :], ri, big), axis=0)
        return jnp.where(ri == pos[None, :], jnp.inf, vv)

    v = lax.fori_loop(0, K - 1, step, v)
    tau_ref[0, :] = jnp.min(v, axis=0)


def _phase2_body(x_ref, tau_ref, qlo_ref, qhi_ref, dt_ref, out_ref):
    c = pl.program_id(0)
    qc = pl.program_id(1)

    @pl.when(qc == 0)
    def _init():
        out_ref[:, :] = jnp.full((BI, 128), -jnp.inf, dtype=jnp.float32)

    qbase = qc * BQ
    active = (qbase < qhi_ref[c]) & (qbase + BQ > qlo_ref[c])

    @pl.when(active)
    def _accum():
        dblk = dt_ref[:, :]
        tau = tau_ref[0, :]
        maskt = dblk <= tau[None, :]
        xq = x_ref[:, :]
        acc = out_ref[:, :]
        for q in range(BQ):
            acc = jnp.maximum(
                acc, jnp.where(maskt[:, q:q + 1], xq[q:q + 1, :], -jnp.inf))
        out_ref[:, :] = acc


def _phase3_body(p_ref, w_ref, b_ref, g_ref, be_ref, out_ref):
    p = p_ref[:, :]
    p = jnp.where(p == -jnp.inf, 0.0, p)
    h = lax.dot_general(p, w_ref[:, :], (((1,), (1,)), ((), ())),
                        preferred_element_type=jnp.float32) + b_ref[0, :][None, :]
    mean = jnp.mean(h, axis=0)
    var = jnp.mean((h - mean[None, :]) ** 2, axis=0)
    h = (h - mean[None, :]) / jnp.sqrt(var + 1e-5) * g_ref[0, :][None, :] \
        + be_ref[0, :][None, :]
    out_ref[:, :] = jnp.maximum(h, 0.0)


@jax.jit
def kernel(x, batch, W, b, gamma, beta):
    n, dim = x.shape
    nq = ((n + CC - 1) // CC) * CC
    g1 = nq // B1
    nch = nq // CC
    gi = nq // BI
    gq = nq // BQ
    xp = jnp.pad(x, ((0, nq - n), (0, 0)))
    bi32 = batch.astype(jnp.int32)
    bp = jnp.pad(bi32, (0, nq - n), constant_values=-1)[None, :]

    seg_lo1 = jnp.searchsorted(bi32, jnp.arange(8, dtype=jnp.int32),
                               side='left').astype(jnp.int32)
    seg_hi1 = jnp.searchsorted(bi32, jnp.arange(8, dtype=jnp.int32),
                               side='right').astype(jnp.int32)
    p0 = jnp.minimum(jnp.arange(g1, dtype=jnp.int32) * B1, n - 1)
    p1 = jnp.minimum(p0 + B1 - 1, n - 1)
    qlo1 = seg_lo1[jnp.clip(bi32[p0], 0, 7)]
    qhi1 = seg_hi1[jnp.clip(bi32[p1], 0, 7)]

    dt, t16 = pl.pallas_call(
        _phase1a_body,
        grid=(g1, nch),
        in_specs=[
            pl.BlockSpec((CC, dim), lambda c, cc: (cc, 0)),
            pl.BlockSpec((1, CC), lambda c, cc: (0, cc)),
            pl.BlockSpec((B1, dim), lambda c, cc: (c, 0)),
            pl.BlockSpec((1, B1), lambda c, cc: (0, c)),
            pl.BlockSpec(memory_space=pltpu.SMEM),
            pl.BlockSpec(memory_space=pltpu.SMEM),
        ],
        out_specs=[
            pl.BlockSpec((CC, B1), lambda c, cc: (cc, c)),
            pl.BlockSpec((1, K, B1), lambda c, cc: (cc, 0, c)),
        ],
        out_shape=[
            jax.ShapeDtypeStruct((nq, nq), jnp.float32),
            jax.ShapeDtypeStruct((nch, K, nq), jnp.float32),
        ],
    )(xp, bp, xp, bp, qlo1, qhi1)

    tau = pl.pallas_call(
        functools.partial(_phase1b_body, nch),
        grid=(g1,),
        in_specs=[pl.BlockSpec((nch, K, B1), lambda c: (0, 0, c))],
        out_specs=pl.BlockSpec((1, B1), lambda c: (0, c)),
        out_shape=jax.ShapeDtypeStruct((1, nq), jnp.float32),
    )(t16)

    seg_lo = jnp.searchsorted(bi32, jnp.arange(8, dtype=jnp.int32),
                              side='left').astype(jnp.int32)
    seg_hi = jnp.searchsorted(bi32, jnp.arange(8, dtype=jnp.int32),
                              side='right').astype(jnp.int32)
    r0 = jnp.minimum(jnp.arange(gi, dtype=jnp.int32) * BI, n - 1)
    r1 = jnp.minimum(r0 + BI - 1, n - 1)
    qlo = seg_lo[jnp.clip(bi32[r0], 0, 7)]
    qhi = seg_hi[jnp.clip(bi32[r1], 0, 7)]

    pool = pl.pallas_call(
        _phase2_body,
        grid=(gi, gq),
        in_specs=[
            pl.BlockSpec((BQ, dim), lambda c, qc: (qc, 0)),
            pl.BlockSpec((1, BQ), lambda c, qc: (0, qc)),
            pl.BlockSpec(memory_space=pltpu.SMEM),
            pl.BlockSpec(memory_space=pltpu.SMEM),
            pl.BlockSpec((BI, BQ), lambda c, qc: (c, qc)),
        ],
        out_specs=pl.BlockSpec((BI, dim), lambda c, qc: (c, 0)),
        out_shape=jax.ShapeDtypeStruct((nq, dim), jnp.float32),
    )(xp, tau, qlo, qhi, dt)

    out = pl.pallas_call(
        _phase3_body,
        in_specs=[
            pl.BlockSpec((n, dim), lambda: (0, 0)),
            pl.BlockSpec((W.shape[0], dim), lambda: (0, 0)),
            pl.BlockSpec((1, W.shape[0]), lambda: (0, 0)),
            pl.BlockSpec((1, W.shape[0]), lambda: (0, 0)),
            pl.BlockSpec((1, W.shape[0]), lambda: (0, 0)),
        ],
        out_specs=pl.BlockSpec((n, W.shape[0]), lambda: (0, 0)),
        out_shape=jax.ShapeDtypeStruct((n, W.shape[0]), jnp.float32),
    )(pool[:n], W, b[None, :], gamma[None, :], beta[None, :])
    return out
